# baseline (speedup 1.0000x reference)
import functools

import jax
import jax.numpy as jnp
from jax import lax
from jax.experimental import pallas as pl
from jax.experimental.pallas import tpu as pltpu
from jax.experimental.pallas import tpu_sc as plsc

N = 10000
NPAD = 10240
IN_DIM = 256
NC, NS = 2, 16
NW = NC * NS
ROWS_PER_TILE = NPAD // NS
E = 160000
B = 128
NCH0 = 57
NCH1 = 22
NCHM = max(NCH0, NCH1)
E0 = NS * B * NCH0
CAP1 = NS * B * NCH1
DEG_W = 16
_f32 = jnp.float32

@functools.cache
def _get_mesh():
    return plsc.VectorSubcoreMesh(core_axis_name="c", subcore_axis_name="s")


def _core_loop(c, step):
    @pl.when(c == 0)
    def _():
        lax.fori_loop(0, NCH0, step, 0)

    @pl.when(c != 0)
    def _():
        lax.fori_loop(0, NCH1, step, 0)


def _deg_body(dsts, ones, out, dst_v, ones_v, accum):
    c = lax.axis_index("c")
    s = lax.axis_index("s")
    pltpu.sync_copy(dsts.at[c, s], dst_v)
    pltpu.sync_copy(ones.at[pl.ds(0, B)], ones_v)
    r0 = s * ROWS_PER_TILE
    pltpu.sync_copy(ones.at[pl.ds(r0, ROWS_PER_TILE)],
                    accum.at[pl.ds(r0, ROWS_PER_TILE)])
    plsc.subcore_barrier()

    def step(j, carry):
        pltpu.sync_copy(ones_v, accum.at[dst_v.at[j]], add=True)
        return carry

    _core_loop(c, step)
    plsc.subcore_barrier()
    pltpu.sync_copy(accum.at[pl.ds(r0, ROWS_PER_TILE)],
                    out.at[c, pl.ds(r0, ROWS_PER_TILE)])


@functools.cache
def _deg_call():
    return pl.kernel(
        _deg_body,
        out_type=jax.ShapeDtypeStruct((NC, NPAD, DEG_W), _f32),
        mesh=_get_mesh(),
        scratch_types=[
            pltpu.VMEM((NCHM, B), jnp.int32),
            pltpu.VMEM((B, DEG_W), _f32),
            pltpu.VMEM_SHARED((NPAD, DEG_W), _f32),
        ],
    )


@functools.cache
def _agg_call(d):
    def body(table, srcs, dsts, out, src_v, dst_v, msgs, accum):
        c = lax.axis_index("c")
        s = lax.axis_index("s")
        pltpu.sync_copy(srcs.at[c, s], src_v)
        pltpu.sync_copy(dsts.at[c, s], dst_v)
        r0 = s * ROWS_PER_TILE
        pltpu.sync_copy(table.at[pl.ds(r0, ROWS_PER_TILE)],
                        accum.at[pl.ds(r0, ROWS_PER_TILE)])
        plsc.subcore_barrier()

        def step(j, carry):
            pltpu.sync_copy(table.at[src_v.at[j]], msgs)
            pltpu.sync_copy(msgs, accum.at[dst_v.at[j]], add=True)
            return carry

        _core_loop(c, step)
        plsc.subcore_barrier()
        pltpu.sync_copy(accum.at[pl.ds(r0, ROWS_PER_TILE)],
                        out.at[c, pl.ds(r0, ROWS_PER_TILE)])

    return pl.kernel(
        body,
        out_type=jax.ShapeDtypeStruct((NC, NPAD, d), _f32),
        mesh=_get_mesh(),
        scratch_types=[
            pltpu.VMEM((NCHM, B), jnp.int32),
            pltpu.VMEM((NCHM, B), jnp.int32),
            pltpu.VMEM((B, d), _f32),
            pltpu.VMEM_SHARED((NPAD, d), _f32),
        ],
    )


def _rows_mask():
    return lax.broadcasted_iota(jnp.int32, (NPAD, 1), 0) < N


def _stage0_body(x_ref, w_ref, pdeg_ref, h_ref, dinv_ref):
    deg = pdeg_ref[0, :, 0:1] + pdeg_ref[1, :, 0:1] - 1.0
    dinv = jnp.where(_rows_mask(), lax.rsqrt(deg), 0.0)
    h = jnp.dot(x_ref[...], w_ref[...], preferred_element_type=_f32)
    h_ref[...] = h * dinv
    dinv_ref[...] = dinv


def _stage_bn_body(p_ref, hprev_ref, dinv_ref, b_ref, g_ref, be_ref, w_ref,
                   out_ref):
    s = p_ref[0] + p_ref[1] - hprev_ref[...]
    dinv = dinv_ref[...]
    u = s * dinv + b_ref[...]
    us = jnp.where(_rows_mask(), u, 0.0)
    m = jnp.sum(us, axis=0, keepdims=True) / N
    v = jnp.sum(us * us, axis=0, keepdims=True) / N - m * m
    y = jnp.maximum((u - m) * lax.rsqrt(v + 1e-5) * g_ref[...] + be_ref[...],
                    0.0)
    out_ref[...] = jnp.dot(y, w_ref[...], preferred_element_type=_f32) * dinv


def _stage3_body(p_ref, hprev_ref, dinv_ref, b_ref, out_ref):
    s = p_ref[0] + p_ref[1] - hprev_ref[...]
    dinv = dinv_ref[...]
    y = jnp.maximum(s * dinv + b_ref[...], 0.0)
    out_ref[...] = y * dinv


def _stage4_body(p_ref, hprev_ref, dinv_ref, w_ref, b_ref, out_ref):
    s = p_ref[0] + p_ref[1] - hprev_ref[...]
    z = jnp.dot(s * dinv_ref[...], w_ref[...],
                preferred_element_type=_f32) + b_ref[...]
    zm = z - jnp.max(z, axis=1, keepdims=True)
    ls = zm - jnp.log(jnp.sum(jnp.exp(zm), axis=1, keepdims=True))
    out_ref[...] = ls[0:N]


def _tc(body, out_shape):
    return pl.pallas_call(body, out_shape=out_shape)


def _padcols(a, cols):
    return jnp.zeros((a.shape[0], cols), _f32).at[:, : a.shape[1]].set(a)


def _padrow(v, cols):
    return jnp.zeros((1, cols), _f32).at[0, : v.shape[0]].set(v)


def kernel(x, edge_index, W1, b1, g1, be1, W2, b2, g2, be2, W3, b3, W4, b4):
    src = edge_index[0]
    dst = edge_index[1]

    def _split(idx):
        a = idx[:E0].reshape(NS, NCH0, B)
        b = jnp.concatenate(
            [idx[E0:], jnp.full((CAP1 - (E - E0),), N, jnp.int32)]
        ).reshape(NS, NCH1, B)
        b = jnp.concatenate(
            [b, jnp.full((NS, NCHM - NCH1, B), N, jnp.int32)], axis=1)
        return jnp.stack([a, b])

    srcs = _split(src)
    dsts = _split(dst)
    ones = jnp.ones((NPAD, DEG_W), _f32)
    x_pad = jnp.zeros((NPAD, IN_DIM), _f32).at[:N].set(x)
    D = 128
    W2p = _padcols(W2, D)
    W3p = _padcols(jnp.zeros((D, W3.shape[1]), _f32).at[:64].set(W3), D)
    W4p = jnp.zeros((D, 2), _f32).at[:32].set(W4)
    b1r, g1r, be1r = _padrow(b1, D), _padrow(g1, D), _padrow(be1, D)
    b2r, g2r, be2r = _padrow(b2, D), _padrow(g2, D), _padrow(be2, D)
    b3r, b4r = _padrow(b3, D), b4.reshape(1, -1)

    pdeg = _deg_call()(dsts, ones)
    h1, dinv = _tc(_stage0_body,
                   (jax.ShapeDtypeStruct((NPAD, D), _f32),
                    jax.ShapeDtypeStruct((NPAD, 1), _f32)))(x_pad, W1, pdeg)
    p1 = _agg_call(D)(h1, srcs, dsts)
    h2 = _tc(_stage_bn_body, jax.ShapeDtypeStruct((NPAD, D), _f32))(
        p1, h1, dinv, b1r, g1r, be1r, W2p)
    p2 = _agg_call(D)(h2, srcs, dsts)
    h3 = _tc(_stage_bn_body, jax.ShapeDtypeStruct((NPAD, D), _f32))(
        p2, h2, dinv, b2r, g2r, be2r, W3p)
    p3 = _agg_call(D)(h3, srcs, dsts)
    h4 = _tc(_stage3_body, jax.ShapeDtypeStruct((NPAD, D), _f32))(
        p3, h3, dinv, b3r)
    p4 = _agg_call(D)(h4, srcs, dsts)
    out = _tc(_stage4_body, jax.ShapeDtypeStruct((N, 2), _f32))(
        p4, h4, dinv, W4p, b4r)
    return out


# speedup vs baseline: 1.1247x; 1.1247x over previous
import functools

import jax
import jax.numpy as jnp
from jax import lax
from jax.experimental import pallas as pl
from jax.experimental.pallas import tpu as pltpu
from jax.experimental.pallas import tpu_sc as plsc

N = 10000
NPAD = 10240
IN_DIM = 256
NC, NS = 2, 16
NW = NC * NS
ROWS_PER_TILE = NPAD // NS
E = 160000
B = 128
NCH0 = 56
NCH1 = 23
NCHM = max(NCH0, NCH1)
E0 = NS * B * NCH0
CAP1 = NS * B * NCH1
DEG_W = 16
_f32 = jnp.float32

@functools.cache
def _get_mesh():
    return plsc.VectorSubcoreMesh(core_axis_name="c", subcore_axis_name="s")


def _core_loop(c, step):
    @pl.when(c == 0)
    def _():
        lax.fori_loop(0, NCH0, step, 0)

    @pl.when(c != 0)
    def _():
        lax.fori_loop(0, NCH1, step, 0)


def _deg_body(dsts, ones, out, dst_v, ones_v, accum):
    c = lax.axis_index("c")
    s = lax.axis_index("s")
    pltpu.sync_copy(dsts.at[c, s], dst_v)
    pltpu.sync_copy(ones.at[pl.ds(0, B)], ones_v)
    r0 = s * ROWS_PER_TILE
    pltpu.sync_copy(ones.at[pl.ds(r0, ROWS_PER_TILE)],
                    accum.at[pl.ds(r0, ROWS_PER_TILE)])
    plsc.subcore_barrier()

    def step(j, carry):
        pltpu.sync_copy(ones_v, accum.at[dst_v.at[j]], add=True)
        return carry

    _core_loop(c, step)
    plsc.subcore_barrier()
    pltpu.sync_copy(accum.at[pl.ds(r0, ROWS_PER_TILE)],
                    out.at[c, pl.ds(r0, ROWS_PER_TILE)])


@functools.cache
def _deg_call():
    return pl.kernel(
        _deg_body,
        out_type=jax.ShapeDtypeStruct((NC, NPAD, DEG_W), _f32),
        mesh=_get_mesh(),
        scratch_types=[
            pltpu.VMEM((NCHM, B), jnp.int32),
            pltpu.VMEM((B, DEG_W), _f32),
            pltpu.VMEM_SHARED((NPAD, DEG_W), _f32),
        ],
    )


@functools.cache
def _agg_call(d):
    def body(table, srcs, dsts, out, src_v, dst_v, m0, m1, accum, g0, g1):
        c = lax.axis_index("c")
        s = lax.axis_index("s")
        pltpu.sync_copy(srcs.at[c, s], src_v)
        pltpu.sync_copy(dsts.at[c, s], dst_v)
        r0 = s * ROWS_PER_TILE
        pltpu.sync_copy(table.at[pl.ds(r0, ROWS_PER_TILE)],
                        accum.at[pl.ds(r0, ROWS_PER_TILE)])
        pltpu.async_copy(table.at[src_v.at[0]], m0, g0)
        plsc.subcore_barrier()

        def make_pair(nch):
            def pair(i, carry):
                j = 2 * i

                @pl.when(j + 1 < nch)
                def _():
                    pltpu.async_copy(table.at[src_v.at[j + 1]], m1, g1)

                pltpu.make_async_copy(table.at[src_v.at[j]], m0, g0).wait()
                pltpu.sync_copy(m0, accum.at[dst_v.at[j]], add=True)

                @pl.when(j + 2 < nch)
                def _():
                    pltpu.async_copy(table.at[src_v.at[j + 2]], m0, g0)

                @pl.when(j + 1 < nch)
                def _():
                    pltpu.make_async_copy(
                        table.at[src_v.at[j + 1]], m1, g1).wait()
                    pltpu.sync_copy(m1, accum.at[dst_v.at[j + 1]], add=True)

                return carry

            return pair

        @pl.when(c == 0)
        def _():
            lax.fori_loop(0, (NCH0 + 1) // 2, make_pair(NCH0), 0)

        @pl.when(c != 0)
        def _():
            lax.fori_loop(0, (NCH1 + 1) // 2, make_pair(NCH1), 0)

        plsc.subcore_barrier()
        pltpu.sync_copy(accum.at[pl.ds(r0, ROWS_PER_TILE)],
                        out.at[c, pl.ds(r0, ROWS_PER_TILE)])

    return pl.kernel(
        body,
        out_type=jax.ShapeDtypeStruct((NC, NPAD, d), _f32),
        mesh=_get_mesh(),
        scratch_types=[
            pltpu.VMEM((NCHM, B), jnp.int32),
            pltpu.VMEM((NCHM, B), jnp.int32),
            pltpu.VMEM((B, d), _f32),
            pltpu.VMEM((B, d), _f32),
            pltpu.VMEM_SHARED((NPAD, d), _f32),
            pltpu.SemaphoreType.DMA,
            pltpu.SemaphoreType.DMA,
        ],
    )


def _rows_mask():
    return lax.broadcasted_iota(jnp.int32, (NPAD, 1), 0) < N


def _stage0_body(x_ref, w_ref, pdeg_ref, h_ref, dinv_ref):
    deg = pdeg_ref[0, :, 0:1] + pdeg_ref[1, :, 0:1] - 1.0
    dinv = jnp.where(_rows_mask(), lax.rsqrt(deg), 0.0)
    h = jnp.dot(x_ref[...], w_ref[...], preferred_element_type=_f32)
    h_ref[...] = h * dinv
    dinv_ref[...] = dinv


def _stage_bn_body(p_ref, hprev_ref, dinv_ref, b_ref, g_ref, be_ref, w_ref,
                   out_ref):
    s = p_ref[0] + p_ref[1] - hprev_ref[...]
    dinv = dinv_ref[...]
    u = s * dinv + b_ref[...]
    us = jnp.where(_rows_mask(), u, 0.0)
    m = jnp.sum(us, axis=0, keepdims=True) / N
    v = jnp.sum(us * us, axis=0, keepdims=True) / N - m * m
    y = jnp.maximum((u - m) * lax.rsqrt(v + 1e-5) * g_ref[...] + be_ref[...],
                    0.0)
    out_ref[...] = jnp.dot(y, w_ref[...], preferred_element_type=_f32) * dinv


def _stage3_body(p_ref, hprev_ref, dinv_ref, b_ref, out_ref):
    s = p_ref[0] + p_ref[1] - hprev_ref[...]
    dinv = dinv_ref[...]
    y = jnp.maximum(s * dinv + b_ref[...], 0.0)
    out_ref[...] = y * dinv


def _stage4_body(p_ref, hprev_ref, dinv_ref, w_ref, b_ref, out_ref):
    s = p_ref[0] + p_ref[1] - hprev_ref[...]
    z = jnp.dot(s * dinv_ref[...], w_ref[...],
                preferred_element_type=_f32) + b_ref[...]
    zm = z - jnp.max(z, axis=1, keepdims=True)
    ls = zm - jnp.log(jnp.sum(jnp.exp(zm), axis=1, keepdims=True))
    out_ref[...] = ls[0:N]


def _tc(body, out_shape):
    return pl.pallas_call(body, out_shape=out_shape)


def _padcols(a, cols):
    return jnp.zeros((a.shape[0], cols), _f32).at[:, : a.shape[1]].set(a)


def _padrow(v, cols):
    return jnp.zeros((1, cols), _f32).at[0, : v.shape[0]].set(v)


def kernel(x, edge_index, W1, b1, g1, be1, W2, b2, g2, be2, W3, b3, W4, b4):
    src = edge_index[0]
    dst = edge_index[1]

    def _split(idx):
        a = idx[:E0].reshape(NS, NCH0, B)
        b = jnp.concatenate(
            [idx[E0:], jnp.full((CAP1 - (E - E0),), N, jnp.int32)]
        ).reshape(NS, NCH1, B)
        b = jnp.concatenate(
            [b, jnp.full((NS, NCHM - NCH1, B), N, jnp.int32)], axis=1)
        return jnp.stack([a, b])

    srcs = _split(src)
    dsts = _split(dst)
    ones = jnp.ones((NPAD, DEG_W), _f32)
    x_pad = jnp.zeros((NPAD, IN_DIM), _f32).at[:N].set(x)
    D = 128
    W2p = _padcols(W2, D)
    W3p = _padcols(jnp.zeros((D, W3.shape[1]), _f32).at[:64].set(W3), D)
    W4p = jnp.zeros((D, 2), _f32).at[:32].set(W4)
    b1r, g1r, be1r = _padrow(b1, D), _padrow(g1, D), _padrow(be1, D)
    b2r, g2r, be2r = _padrow(b2, D), _padrow(g2, D), _padrow(be2, D)
    b3r, b4r = _padrow(b3, D), b4.reshape(1, -1)

    pdeg = _deg_call()(dsts, ones)
    h1, dinv = _tc(_stage0_body,
                   (jax.ShapeDtypeStruct((NPAD, D), _f32),
                    jax.ShapeDtypeStruct((NPAD, 1), _f32)))(x_pad, W1, pdeg)
    p1 = _agg_call(D)(h1, srcs, dsts)
    h2 = _tc(_stage_bn_body, jax.ShapeDtypeStruct((NPAD, D), _f32))(
        p1, h1, dinv, b1r, g1r, be1r, W2p)
    p2 = _agg_call(D)(h2, srcs, dsts)
    h3 = _tc(_stage_bn_body, jax.ShapeDtypeStruct((NPAD, D), _f32))(
        p2, h2, dinv, b2r, g2r, be2r, W3p)
    p3 = _agg_call(D)(h3, srcs, dsts)
    h4 = _tc(_stage3_body, jax.ShapeDtypeStruct((NPAD, D), _f32))(
        p3, h3, dinv, b3r)
    p4 = _agg_call(D)(h4, srcs, dsts)
    out = _tc(_stage4_body, jax.ShapeDtypeStruct((N, 2), _f32))(
        p4, h4, dinv, W4p, b4r)
    return out


# speedup vs baseline: 1.1405x; 1.0141x over previous
import functools

import jax
import jax.numpy as jnp
from jax import lax
from jax.experimental import pallas as pl
from jax.experimental.pallas import tpu as pltpu
from jax.experimental.pallas import tpu_sc as plsc

N = 10000
NPAD = 10240
IN_DIM = 256
NC, NS = 2, 16
NW = NC * NS
ROWS_PER_TILE = NPAD // NS
E = 160000
B = 128
NCH0 = 62
NCH1 = 17
NCHM = max(NCH0, NCH1)
E0 = NS * B * NCH0
CAP1 = NS * B * NCH1
DEG_W = 16
_f32 = jnp.float32

@functools.cache
def _get_mesh():
    return plsc.VectorSubcoreMesh(core_axis_name="c", subcore_axis_name="s")


def _core_loop(c, step):
    @pl.when(c == 0)
    def _():
        lax.fori_loop(0, NCH0, step, 0)

    @pl.when(c != 0)
    def _():
        lax.fori_loop(0, NCH1, step, 0)


def _deg_body(dsts, ones, out, dst_v, ones_v, accum):
    c = lax.axis_index("c")
    s = lax.axis_index("s")
    pltpu.sync_copy(dsts.at[c, s], dst_v)
    pltpu.sync_copy(ones.at[pl.ds(0, B)], ones_v)
    r0 = s * ROWS_PER_TILE
    pltpu.sync_copy(ones.at[pl.ds(r0, ROWS_PER_TILE)],
                    accum.at[pl.ds(r0, ROWS_PER_TILE)])
    plsc.subcore_barrier()

    def step(j, carry):
        pltpu.sync_copy(ones_v, accum.at[dst_v.at[j]], add=True)
        return carry

    _core_loop(c, step)
    plsc.subcore_barrier()
    pltpu.sync_copy(accum.at[pl.ds(r0, ROWS_PER_TILE)],
                    out.at[c, pl.ds(r0, ROWS_PER_TILE)])


@functools.cache
def _deg_call():
    return pl.kernel(
        _deg_body,
        out_type=jax.ShapeDtypeStruct((NC, NPAD, DEG_W), _f32),
        mesh=_get_mesh(),
        scratch_types=[
            pltpu.VMEM((NCHM, B), jnp.int32),
            pltpu.VMEM((B, DEG_W), _f32),
            pltpu.VMEM_SHARED((NPAD, DEG_W), _f32),
        ],
    )


@functools.cache
def _agg_call(d):
    def body(table, srcs, dsts, out, src_v, dst_v, m0, m1, accum, g0, g1):
        c = lax.axis_index("c")
        s = lax.axis_index("s")
        pltpu.sync_copy(srcs.at[c, s], src_v)
        pltpu.sync_copy(dsts.at[c, s], dst_v)
        r0 = s * ROWS_PER_TILE
        pltpu.sync_copy(table.at[pl.ds(r0, ROWS_PER_TILE)],
                        accum.at[pl.ds(r0, ROWS_PER_TILE)])
        pltpu.async_copy(table.at[src_v.at[0]], m0, g0)
        plsc.subcore_barrier()

        def make_pair(nch):
            def pair(i, carry):
                j = 2 * i

                @pl.when(j + 1 < nch)
                def _():
                    pltpu.async_copy(table.at[src_v.at[j + 1]], m1, g1)

                pltpu.make_async_copy(table.at[src_v.at[j]], m0, g0).wait()
                pltpu.sync_copy(m0, accum.at[dst_v.at[j]], add=True)

                @pl.when(j + 2 < nch)
                def _():
                    pltpu.async_copy(table.at[src_v.at[j + 2]], m0, g0)

                @pl.when(j + 1 < nch)
                def _():
                    pltpu.make_async_copy(
                        table.at[src_v.at[j + 1]], m1, g1).wait()
                    pltpu.sync_copy(m1, accum.at[dst_v.at[j + 1]], add=True)

                return carry

            return pair

        @pl.when(c == 0)
        def _():
            lax.fori_loop(0, (NCH0 + 1) // 2, make_pair(NCH0), 0)

        @pl.when(c != 0)
        def _():
            lax.fori_loop(0, (NCH1 + 1) // 2, make_pair(NCH1), 0)

        plsc.subcore_barrier()
        pltpu.sync_copy(accum.at[pl.ds(r0, ROWS_PER_TILE)],
                        out.at[c, pl.ds(r0, ROWS_PER_TILE)])

    return pl.kernel(
        body,
        out_type=jax.ShapeDtypeStruct((NC, NPAD, d), _f32),
        mesh=_get_mesh(),
        scratch_types=[
            pltpu.VMEM((NCHM, B), jnp.int32),
            pltpu.VMEM((NCHM, B), jnp.int32),
            pltpu.VMEM((B, d), _f32),
            pltpu.VMEM((B, d), _f32),
            pltpu.VMEM_SHARED((NPAD, d), _f32),
            pltpu.SemaphoreType.DMA,
            pltpu.SemaphoreType.DMA,
        ],
    )


def _rows_mask():
    return lax.broadcasted_iota(jnp.int32, (NPAD, 1), 0) < N


def _stage0_body(x_ref, w_ref, pdeg_ref, h_ref, dinv_ref):
    deg = pdeg_ref[0, :, 0:1] + pdeg_ref[1, :, 0:1] - 1.0
    dinv = jnp.where(_rows_mask(), lax.rsqrt(deg), 0.0)
    h = jnp.dot(x_ref[...], w_ref[...], preferred_element_type=_f32)
    h_ref[...] = h * dinv
    dinv_ref[...] = dinv


def _stage_bn_body(p_ref, hprev_ref, dinv_ref, b_ref, g_ref, be_ref, w_ref,
                   out_ref):
    s = p_ref[0] + p_ref[1] - hprev_ref[...]
    dinv = dinv_ref[...]
    u = s * dinv + b_ref[...]
    us = jnp.where(_rows_mask(), u, 0.0)
    m = jnp.sum(us, axis=0, keepdims=True) / N
    v = jnp.sum(us * us, axis=0, keepdims=True) / N - m * m
    y = jnp.maximum((u - m) * lax.rsqrt(v + 1e-5) * g_ref[...] + be_ref[...],
                    0.0)
    out_ref[...] = jnp.dot(y, w_ref[...], preferred_element_type=_f32) * dinv


def _stage3_body(p_ref, hprev_ref, dinv_ref, b_ref, out_ref):
    s = p_ref[0] + p_ref[1] - hprev_ref[...]
    dinv = dinv_ref[...]
    y = jnp.maximum(s * dinv + b_ref[...], 0.0)
    out_ref[...] = y * dinv


def _stage4_body(p_ref, hprev_ref, dinv_ref, w_ref, b_ref, out_ref):
    s = p_ref[0] + p_ref[1] - hprev_ref[...]
    z = jnp.dot(s * dinv_ref[...], w_ref[...],
                preferred_element_type=_f32) + b_ref[...]
    zm = z - jnp.max(z, axis=1, keepdims=True)
    ls = zm - jnp.log(jnp.sum(jnp.exp(zm), axis=1, keepdims=True))
    out_ref[...] = ls[0:N]


def _tc(body, out_shape):
    return pl.pallas_call(body, out_shape=out_shape)


def _padcols(a, cols):
    return jnp.zeros((a.shape[0], cols), _f32).at[:, : a.shape[1]].set(a)


def _padrow(v, cols):
    return jnp.zeros((1, cols), _f32).at[0, : v.shape[0]].set(v)


def kernel(x, edge_index, W1, b1, g1, be1, W2, b2, g2, be2, W3, b3, W4, b4):
    src = edge_index[0]
    dst = edge_index[1]

    def _split(idx):
        a = idx[:E0].reshape(NS, NCH0, B)
        b = jnp.concatenate(
            [idx[E0:], jnp.full((CAP1 - (E - E0),), N, jnp.int32)]
        ).reshape(NS, NCH1, B)
        b = jnp.concatenate(
            [b, jnp.full((NS, NCHM - NCH1, B), N, jnp.int32)], axis=1)
        return jnp.stack([a, b])

    srcs = _split(src)
    dsts = _split(dst)
    ones = jnp.ones((NPAD, DEG_W), _f32)
    x_pad = jnp.zeros((NPAD, IN_DIM), _f32).at[:N].set(x)
    D = 128
    W2p = _padcols(W2, D)
    W3p = _padcols(jnp.zeros((D, W3.shape[1]), _f32).at[:64].set(W3), D)
    W4p = jnp.zeros((D, 2), _f32).at[:32].set(W4)
    b1r, g1r, be1r = _padrow(b1, D), _padrow(g1, D), _padrow(be1, D)
    b2r, g2r, be2r = _padrow(b2, D), _padrow(g2, D), _padrow(be2, D)
    b3r, b4r = _padrow(b3, D), b4.reshape(1, -1)

    pdeg = _deg_call()(dsts, ones)
    h1, dinv = _tc(_stage0_body,
                   (jax.ShapeDtypeStruct((NPAD, D), _f32),
                    jax.ShapeDtypeStruct((NPAD, 1), _f32)))(x_pad, W1, pdeg)
    p1 = _agg_call(D)(h1, srcs, dsts)
    h2 = _tc(_stage_bn_body, jax.ShapeDtypeStruct((NPAD, D), _f32))(
        p1, h1, dinv, b1r, g1r, be1r, W2p)
    p2 = _agg_call(D)(h2, srcs, dsts)
    h3 = _tc(_stage_bn_body, jax.ShapeDtypeStruct((NPAD, D), _f32))(
        p2, h2, dinv, b2r, g2r, be2r, W3p)
    p3 = _agg_call(D)(h3, srcs, dsts)
    h4 = _tc(_stage3_body, jax.ShapeDtypeStruct((NPAD, D), _f32))(
        p3, h3, dinv, b3r)
    p4 = _agg_call(D)(h4, srcs, dsts)
    out = _tc(_stage4_body, jax.ShapeDtypeStruct((N, 2), _f32))(
        p4, h4, dinv, W4p, b4r)
    return out
